# baseline (speedup 1.0000x reference)
import functools

import jax
import jax.numpy as jnp
from jax import lax
from jax.experimental import pallas as pl
from jax.experimental.pallas import tpu as pltpu
from jax.experimental.pallas import tpu_sc as plsc

NC, NS, L = 2, 16, 16
NW = NC * NS
CH = 128

_LOG1P = (-0.01741407752432069, 0.08269123711159879, -0.19035433673328145,
          0.3157473167581094, -0.4973732161579986, 0.9998476974962415,
          1.4720650111430504e-06)


def _softplus(v):
    u = jnp.exp(-jnp.abs(v))
    p = jnp.full_like(v, _LOG1P[0])
    for c in _LOG1P[1:]:
        p = p * u + c
    return jnp.maximum(v, 0.0) + p


def _sigmoid(v):
    return 1.0 / (1.0 + jnp.exp(-v))


def _edge_body(psrc, pdst, srch, dsth, mean_h, var_h, wt_h,
               sidx_all, didx_all, A0, A1, A2, B0, B1, B2, M0, M1, M2,
               V0, V1, V2, W0, W1, W2,
               semA0, semA1, semA2, semB0, semB1, semB2,
               semO0, semO1, semO2):
    c = lax.axis_index("c")
    s = lax.axis_index("s")
    wid = s * NC + c
    n_chunks_total = mean_h.shape[0] // CH
    base = n_chunks_total // NW
    rem = n_chunks_total % NW
    ncmax = base + (1 if rem else 0)
    start = wid * base + jnp.minimum(wid, rem)
    n = base + jnp.where(wid < rem, 1, 0)
    lanes = jnp.arange(L, dtype=jnp.int32)

    A = (A0, A1, A2)
    B = (B0, B1, B2)
    M = (M0, M1, M2)
    V = (V0, V1, V2)
    W = (W0, W1, W2)
    semA = (semA0, semA1, semA2)
    semB = (semB0, semB1, semB2)
    semO = (semO0, semO1, semO2)
    DEPTH = 3

    pltpu.sync_copy(srch.at[pl.ds(start * CH, base * CH)],
                    sidx_all.at[pl.ds(0, base * CH)])
    pltpu.sync_copy(dsth.at[pl.ds(start * CH, base * CH)],
                    didx_all.at[pl.ds(0, base * CH)])

    @pl.when(wid < rem)
    def _():
        pltpu.sync_copy(srch.at[pl.ds(start * CH + base * CH, CH)],
                        sidx_all.at[pl.ds(base * CH, CH)])
        pltpu.sync_copy(dsth.at[pl.ds(start * CH + base * CH, CH)],
                        didx_all.at[pl.ds(base * CH, CH)])

    def gathers(i, b):
        o = i * CH
        return (
            pltpu.make_async_copy(psrc.at[sidx_all.at[pl.ds(o, CH)]],
                                  A[b], semA[b]),
            pltpu.make_async_copy(pdst.at[didx_all.at[pl.ds(o, CH)]],
                                  B[b], semB[b]),
        )

    def outs(i, b):
        off = (start + i) * CH
        return (
            pltpu.make_async_copy(M[b], mean_h.at[pl.ds(off, CH)], semO[b]),
            pltpu.make_async_copy(V[b], var_h.at[pl.ds(off, CH)], semO[b]),
            pltpu.make_async_copy(W[b], wt_h.at[pl.ds(off, CH)], semO[b]),
        )

    def compute(i, b):
        Ab, Bb, Mb, Vb, Wb = A[b], B[b], M[b], V[b], W[b]

        @plsc.parallel_loop(0, CH, unroll=2,
                            carry=jnp.zeros((L,), jnp.float32))
        def erow(e, wacc):
            m0 = Ab[e, pl.ds(0, L)] + Bb[e, pl.ds(0, L)]
            m1 = Ab[e, pl.ds(16, L)] + Bb[e, pl.ds(16, L)]
            Mb[e, pl.ds(0, L)] = m0
            Mb[e, pl.ds(16, L)] = m1
            v0 = Ab[e, pl.ds(32, L)] + Bb[e, pl.ds(32, L)]
            v1 = Ab[e, pl.ds(48, L)] + Bb[e, pl.ds(48, L)]
            Vb[e, pl.ds(0, L)] = _softplus(v0) + 1e-6
            Vb[e, pl.ds(16, L)] = _softplus(v1) + 1e-6
            w = _sigmoid(Ab[e, pl.ds(64, L)] + Bb[e, pl.ds(64, L)])
            lane = e % L
            wacc = jnp.where(lanes == lane, w, wacc)

            @pl.when(lane == L - 1)
            def _():
                Wb[pl.ds(e - (L - 1), L)] = wacc

            return wacc

        for cp in outs(i, b):
            cp.start()

    def step(i, b):
        for cp in gathers(i, b):
            cp.wait()

        @pl.when(i + DEPTH - 1 < n)
        def _():
            for cp in gathers(i + DEPTH - 1, (b + DEPTH - 1) % DEPTH):
                cp.start()

        @pl.when(i >= DEPTH)
        def _():
            for cp in outs(i - DEPTH, b):
                cp.wait()

        compute(i, b)

    for j in range(DEPTH - 1):
        for cp in gathers(j, j):
            cp.start()

    def body(i, carry):
        for b in range(DEPTH):
            @pl.when(i % DEPTH == b)
            def _(b=b):
                step(i, b)

        return carry

    lax.fori_loop(0, n, body, 0)

    def drain(i):
        for b in range(DEPTH):
            @pl.when(i % DEPTH == b)
            def _(b=b):
                for cp in outs(i, b):
                    cp.wait()

    for j in range(DEPTH):
        drain(n - DEPTH + j)


def _edge_stage(psrc, pdst, src, dst):
    e = src.shape[0]
    ncmax = (e // CH + NW - 1) // NW
    k = pl.kernel(
        _edge_body,
        out_type=(
            jax.ShapeDtypeStruct((e, 32), jnp.float32),
            jax.ShapeDtypeStruct((e, 32), jnp.float32),
            jax.ShapeDtypeStruct((e,), jnp.float32),
        ),
        mesh=plsc.VectorSubcoreMesh(core_axis_name="c", subcore_axis_name="s"),
        compiler_params=pltpu.CompilerParams(use_tc_tiling_on_sc=False),
        scratch_types=[
            pltpu.VMEM((ncmax * CH,), jnp.int32),
            pltpu.VMEM((ncmax * CH,), jnp.int32),
            pltpu.VMEM((CH, 80), jnp.float32),
            pltpu.VMEM((CH, 80), jnp.float32),
            pltpu.VMEM((CH, 80), jnp.float32),
            pltpu.VMEM((CH, 80), jnp.float32),
            pltpu.VMEM((CH, 80), jnp.float32),
            pltpu.VMEM((CH, 80), jnp.float32),
            pltpu.VMEM((CH, 32), jnp.float32),
            pltpu.VMEM((CH, 32), jnp.float32),
            pltpu.VMEM((CH, 32), jnp.float32),
            pltpu.VMEM((CH, 32), jnp.float32),
            pltpu.VMEM((CH, 32), jnp.float32),
            pltpu.VMEM((CH, 32), jnp.float32),
            pltpu.VMEM((CH,), jnp.float32),
            pltpu.VMEM((CH,), jnp.float32),
            pltpu.VMEM((CH,), jnp.float32),
            pltpu.SemaphoreType.DMA,
            pltpu.SemaphoreType.DMA,
            pltpu.SemaphoreType.DMA,
            pltpu.SemaphoreType.DMA,
            pltpu.SemaphoreType.DMA,
            pltpu.SemaphoreType.DMA,
            pltpu.SemaphoreType.DMA,
            pltpu.SemaphoreType.DMA,
            pltpu.SemaphoreType.DMA,
        ],
    )
    return k(psrc, pdst, src, dst)


def _agg_body(xh, srch, dsth, psum_h, pcnt_h,
              sidx0, sidx1, didx0, didx1, rows0, rows1, ones, acc, cntsh,
              semG0, semG1):
    c = lax.axis_index("c")
    s = lax.axis_index("s")
    wid = s * NC + c
    n_nodes = acc.shape[0]
    n_chunks_total = srch.shape[0] // CH
    base = n_chunks_total // NW
    rem = n_chunks_total % NW
    start = wid * base + jnp.minimum(wid, rem)
    n = base + jnp.where(wid < rem, 1, 0)
    rows = (rows0, rows1)
    didx = (didx0, didx1)
    semG = (semG0, semG1)

    zero = jnp.zeros((L,), jnp.float32)

    @plsc.parallel_loop(0, CH)
    def zrow(r):
        for j in range(128 // L):
            rows0[r, pl.ds(j * L, L)] = zero
        ones[r, pl.ds(0, L)] = zero

    rpt = n_nodes // NS
    nslc = rpt // 125
    for j in range(nslc):
        r0 = s * rpt + j * 125
        pltpu.sync_copy(rows0.at[pl.ds(0, 125)], acc.at[pl.ds(r0, 125)])
        pltpu.sync_copy(ones.at[pl.ds(0, 125)], cntsh.at[pl.ds(r0, 125)])
    plsc.subcore_barrier()

    one = jnp.full((L,), 1.0, jnp.float32)

    @plsc.parallel_loop(0, CH)
    def orow(r):
        ones[r, pl.ds(0, L)] = one

    sidx = (sidx0, sidx1)

    def gath(i, b):
        return pltpu.make_async_copy(xh.at[sidx[b]], rows[b], semG[b])

    def stage_idx(i, b):
        pltpu.sync_copy(srch.at[pl.ds((start + i) * CH, CH)], sidx[b])
        pltpu.sync_copy(dsth.at[pl.ds((start + i) * CH, CH)], didx[b])

    def step(i, b, nb):
        gath(i, b).wait()

        @pl.when(i + 1 < n)
        def _():
            stage_idx(i + 1, nb)
            gath(i + 1, nb).start()

        pltpu.sync_copy(rows[b], acc.at[didx[b]], add=True)
        pltpu.sync_copy(ones, cntsh.at[didx[b]], add=True)

    stage_idx(0, 0)
    gath(0, 0).start()

    def body(i, carry):
        @pl.when(i % 2 == 0)
        def _():
            step(i, 0, 1)

        @pl.when(i % 2 == 1)
        def _():
            step(i, 1, 0)

        return carry

    lax.fori_loop(0, n, body, 0)
    plsc.subcore_barrier()

    for j in range(nslc):
        r0 = s * rpt + j * 125
        ro = c * n_nodes + r0
        pltpu.sync_copy(acc.at[pl.ds(r0, 125)], rows0.at[pl.ds(0, 125)])
        pltpu.sync_copy(rows0.at[pl.ds(0, 125)], psum_h.at[pl.ds(ro, 125)])
        pltpu.sync_copy(cntsh.at[pl.ds(r0, 125)], ones.at[pl.ds(0, 125)])
        pltpu.sync_copy(ones.at[pl.ds(0, 125)], pcnt_h.at[pl.ds(ro, 125)])


def _agg_stage(x, src, dst):
    n_nodes = x.shape[0]
    k = pl.kernel(
        _agg_body,
        out_type=(
            jax.ShapeDtypeStruct((NC * n_nodes, 128), jnp.float32),
            jax.ShapeDtypeStruct((NC * n_nodes, 16), jnp.float32),
        ),
        mesh=plsc.VectorSubcoreMesh(core_axis_name="c", subcore_axis_name="s"),
        compiler_params=pltpu.CompilerParams(use_tc_tiling_on_sc=False),
        scratch_types=[
            pltpu.VMEM((CH,), jnp.int32),
            pltpu.VMEM((CH,), jnp.int32),
            pltpu.VMEM((CH,), jnp.int32),
            pltpu.VMEM((CH,), jnp.int32),
            pltpu.VMEM((CH, 128), jnp.float32),
            pltpu.VMEM((CH, 128), jnp.float32),
            pltpu.VMEM((CH, 16), jnp.float32),
            pltpu.VMEM_SHARED((n_nodes, 128), jnp.float32),
            pltpu.VMEM_SHARED((n_nodes, 16), jnp.float32),
            pltpu.SemaphoreType.DMA,
            pltpu.SemaphoreType.DMA,
        ],
    )
    return k(x, src, dst)


def _stage_b_body(psum_ref, pcnt_ref, gnnW_ref, gnnb_ref, wsrc_ref, wdst_ref,
                  zb_ref, psrc_ref, pdst_ref):
    n = psum_ref.shape[0] // 2
    summed = psum_ref[0:n] + psum_ref[n:2 * n]
    pcnt = pcnt_ref[0:n, 0:1] + pcnt_ref[n:2 * n, 0:1]
    cnt = jnp.maximum(pcnt, 1.0)
    agg = summed / cnt
    h = jnp.maximum(
        jnp.dot(agg, gnnW_ref[...].T, preferred_element_type=jnp.float32)
        + gnnb_ref[...], 0.0)
    psrc_ref[...] = (
        jnp.dot(h, wsrc_ref[...].T, preferred_element_type=jnp.float32)
        + zb_ref[...])
    pdst_ref[...] = jnp.dot(h, wdst_ref[...].T,
                            preferred_element_type=jnp.float32)


def _stage_b(psum, pcnt, gnn_W, gnn_b, wsrc, wdst, zb):
    n = psum.shape[0] // 2
    return pl.pallas_call(
        _stage_b_body,
        out_shape=(
            jax.ShapeDtypeStruct((n, 80), jnp.float32),
            jax.ShapeDtypeStruct((n, 80), jnp.float32),
        ),
    )(psum, pcnt, gnn_W, gnn_b, wsrc, wdst, zb)


def kernel(x_topology, x_temporal, edge_index, gnn_W, gnn_b, mean_W, mean_b,
           var_W, var_b, weight_W, weight_b):
    B, N, _ = x_topology.shape
    n_nodes = B * N
    x = jnp.concatenate([x_topology, x_temporal], axis=-1).reshape(
        n_nodes, -1).astype(jnp.float32)
    src = edge_index[0]
    dst = edge_index[1]

    psum, pcnt = _agg_stage(x, src, dst)

    wsrc = jnp.concatenate([mean_W[:, :128], var_W[:, :128],
                            jnp.tile(weight_W[:, :128], (16, 1))], axis=0)
    wdst = jnp.concatenate([mean_W[:, 128:], var_W[:, 128:],
                            jnp.tile(weight_W[:, 128:], (16, 1))], axis=0)
    zb = jnp.concatenate([mean_b, var_b, jnp.tile(weight_b, 16)])[None, :]

    psrc, pdst = _stage_b(psum, pcnt, gnn_W, gnn_b[None, :], wsrc, wdst, zb)
    mean, var, wt = _edge_stage(psrc, pdst, src, dst)
    return (mean, var, wt)


# speedup vs baseline: 1.0009x; 1.0009x over previous
import functools

import jax
import jax.numpy as jnp
from jax import lax
from jax.experimental import pallas as pl
from jax.experimental.pallas import tpu as pltpu
from jax.experimental.pallas import tpu_sc as plsc

NC, NS, L = 2, 16, 16
NW = NC * NS
CH = 128

_LOG1P = (-0.01741407752432069, 0.08269123711159879, -0.19035433673328145,
          0.3157473167581094, -0.4973732161579986, 0.9998476974962415,
          1.4720650111430504e-06)


def _softplus(v):
    u = jnp.exp(-jnp.abs(v))
    p = jnp.full_like(v, _LOG1P[0])
    for c in _LOG1P[1:]:
        p = p * u + c
    return jnp.maximum(v, 0.0) + p


def _sigmoid(v):
    return 1.0 / (1.0 + jnp.exp(-v))


def _edge_body(psrc, pdst, srch, dsth, mean_h, var_h, wt_h,
               sidx_all, didx_all, A0, A1, B0, B1, M0, M1, V0, V1, W0, W1,
               semA0, semA1, semB0, semB1, semO0, semO1):
    c = lax.axis_index("c")
    s = lax.axis_index("s")
    wid = s * NC + c
    n_chunks_total = mean_h.shape[0] // CH
    base = n_chunks_total // NW
    rem = n_chunks_total % NW
    ncmax = base + (1 if rem else 0)
    start = wid * base + jnp.minimum(wid, rem)
    n = base + jnp.where(wid < rem, 1, 0)
    lanes = jnp.arange(L, dtype=jnp.int32)

    A = (A0, A1)
    B = (B0, B1)
    M = (M0, M1)
    V = (V0, V1)
    W = (W0, W1)
    semA = (semA0, semA1)
    semB = (semB0, semB1)
    semO = (semO0, semO1)

    pltpu.sync_copy(srch.at[pl.ds(start * CH, base * CH)],
                    sidx_all.at[pl.ds(0, base * CH)])
    pltpu.sync_copy(dsth.at[pl.ds(start * CH, base * CH)],
                    didx_all.at[pl.ds(0, base * CH)])

    @pl.when(wid < rem)
    def _():
        pltpu.sync_copy(srch.at[pl.ds(start * CH + base * CH, CH)],
                        sidx_all.at[pl.ds(base * CH, CH)])
        pltpu.sync_copy(dsth.at[pl.ds(start * CH + base * CH, CH)],
                        didx_all.at[pl.ds(base * CH, CH)])

    def gathers(i, b):
        o = i * CH
        return (
            pltpu.make_async_copy(psrc.at[sidx_all.at[pl.ds(o, CH)]],
                                  A[b], semA[b]),
            pltpu.make_async_copy(pdst.at[didx_all.at[pl.ds(o, CH)]],
                                  B[b], semB[b]),
        )

    def outs(i, b):
        off = (start + i) * CH
        return (
            pltpu.make_async_copy(M[b], mean_h.at[pl.ds(off, CH)], semO[b]),
            pltpu.make_async_copy(V[b], var_h.at[pl.ds(off, CH)], semO[b]),
            pltpu.make_async_copy(W[b], wt_h.at[pl.ds(off, CH)], semO[b]),
        )

    def compute(i, b):
        Ab, Bb, Mb, Vb, Wb = A[b], B[b], M[b], V[b], W[b]

        @plsc.parallel_loop(0, CH, unroll=2,
                            carry=jnp.zeros((L,), jnp.float32))
        def erow(e, wacc):
            m0 = Ab[e, pl.ds(0, L)] + Bb[e, pl.ds(0, L)]
            m1 = Ab[e, pl.ds(16, L)] + Bb[e, pl.ds(16, L)]
            Mb[e, pl.ds(0, L)] = m0
            Mb[e, pl.ds(16, L)] = m1
            v0 = Ab[e, pl.ds(32, L)] + Bb[e, pl.ds(32, L)]
            v1 = Ab[e, pl.ds(48, L)] + Bb[e, pl.ds(48, L)]
            Vb[e, pl.ds(0, L)] = _softplus(v0) + 1e-6
            Vb[e, pl.ds(16, L)] = _softplus(v1) + 1e-6
            w = _sigmoid(Ab[e, pl.ds(64, L)] + Bb[e, pl.ds(64, L)])
            lane = e % L
            wacc = jnp.where(lanes == lane, w, wacc)

            @pl.when(lane == L - 1)
            def _():
                Wb[pl.ds(e - (L - 1), L)] = wacc

            return wacc

        for cp in outs(i, b):
            cp.start()

    def step(i, b, nb):
        for cp in gathers(i, b):
            cp.wait()

        @pl.when(i + 1 < n)
        def _():
            for cp in gathers(i + 1, nb):
                cp.start()

        @pl.when(i >= 2)
        def _():
            for cp in outs(i - 2, b):
                cp.wait()

        compute(i, b)

    for cp in gathers(0, 0):
        cp.start()

    def body(i, carry):
        @pl.when(i % 2 == 0)
        def _():
            step(i, 0, 1)

        @pl.when(i % 2 == 1)
        def _():
            step(i, 1, 0)

        return carry

    lax.fori_loop(0, n, body, 0)

    def drain(i):
        @pl.when(i % 2 == 0)
        def _():
            for cp in outs(i, 0):
                cp.wait()

        @pl.when(i % 2 == 1)
        def _():
            for cp in outs(i, 1):
                cp.wait()

    drain(n - 2)
    drain(n - 1)


def _edge_stage(psrc, pdst, src, dst):
    e = src.shape[0]
    ncmax = (e // CH + NW - 1) // NW
    k = pl.kernel(
        _edge_body,
        out_type=(
            jax.ShapeDtypeStruct((e, 32), jnp.float32),
            jax.ShapeDtypeStruct((e, 32), jnp.float32),
            jax.ShapeDtypeStruct((e,), jnp.float32),
        ),
        mesh=plsc.VectorSubcoreMesh(core_axis_name="c", subcore_axis_name="s"),
        compiler_params=pltpu.CompilerParams(use_tc_tiling_on_sc=False),
        scratch_types=[
            pltpu.VMEM((ncmax * CH,), jnp.int32),
            pltpu.VMEM((ncmax * CH,), jnp.int32),
            pltpu.VMEM((CH, 80), jnp.float32),
            pltpu.VMEM((CH, 80), jnp.float32),
            pltpu.VMEM((CH, 80), jnp.float32),
            pltpu.VMEM((CH, 80), jnp.float32),
            pltpu.VMEM((CH, 32), jnp.float32),
            pltpu.VMEM((CH, 32), jnp.float32),
            pltpu.VMEM((CH, 32), jnp.float32),
            pltpu.VMEM((CH, 32), jnp.float32),
            pltpu.VMEM((CH,), jnp.float32),
            pltpu.VMEM((CH,), jnp.float32),
            pltpu.SemaphoreType.DMA,
            pltpu.SemaphoreType.DMA,
            pltpu.SemaphoreType.DMA,
            pltpu.SemaphoreType.DMA,
            pltpu.SemaphoreType.DMA,
            pltpu.SemaphoreType.DMA,
        ],
    )
    return k(psrc, pdst, src, dst)


def _agg_body(xh, srch, dsth, psum_h, pcnt_h,
              sidx0, sidx1, didx0, didx1, rows0, rows1, ones, acc, cntsh,
              semG0, semG1):
    c = lax.axis_index("c")
    s = lax.axis_index("s")
    wid = s * NC + c
    n_nodes = acc.shape[0]
    n_chunks_total = srch.shape[0] // CH
    base = n_chunks_total // NW
    rem = n_chunks_total % NW
    start = wid * base + jnp.minimum(wid, rem)
    n = base + jnp.where(wid < rem, 1, 0)
    rows = (rows0, rows1)
    didx = (didx0, didx1)
    semG = (semG0, semG1)

    zero = jnp.zeros((L,), jnp.float32)

    @plsc.parallel_loop(0, CH)
    def zrow(r):
        for j in range(128 // L):
            rows0[r, pl.ds(j * L, L)] = zero
        ones[r, pl.ds(0, L)] = zero

    rpt = n_nodes // NS
    nslc = rpt // 125
    for j in range(nslc):
        r0 = s * rpt + j * 125
        pltpu.sync_copy(rows0.at[pl.ds(0, 125)], acc.at[pl.ds(r0, 125)])
        pltpu.sync_copy(ones.at[pl.ds(0, 125)], cntsh.at[pl.ds(r0, 125)])
    plsc.subcore_barrier()

    one = jnp.full((L,), 1.0, jnp.float32)

    @plsc.parallel_loop(0, CH)
    def orow(r):
        ones[r, pl.ds(0, L)] = one

    sidx = (sidx0, sidx1)

    def gath(i, b):
        return pltpu.make_async_copy(xh.at[sidx[b]], rows[b], semG[b])

    def stage_idx(i, b):
        pltpu.sync_copy(srch.at[pl.ds((start + i) * CH, CH)], sidx[b])
        pltpu.sync_copy(dsth.at[pl.ds((start + i) * CH, CH)], didx[b])

    def step(i, b, nb):
        gath(i, b).wait()

        @pl.when(i + 1 < n)
        def _():
            stage_idx(i + 1, nb)
            gath(i + 1, nb).start()

        pltpu.sync_copy(rows[b], acc.at[didx[b]], add=True)
        pltpu.sync_copy(ones, cntsh.at[didx[b]], add=True)

    stage_idx(0, 0)
    gath(0, 0).start()

    def body(i, carry):
        @pl.when(i % 2 == 0)
        def _():
            step(i, 0, 1)

        @pl.when(i % 2 == 1)
        def _():
            step(i, 1, 0)

        return carry

    lax.fori_loop(0, n, body, 0)
    plsc.subcore_barrier()

    for j in range(nslc):
        r0 = s * rpt + j * 125
        ro = c * n_nodes + r0
        pltpu.sync_copy(acc.at[pl.ds(r0, 125)], rows0.at[pl.ds(0, 125)])
        pltpu.sync_copy(rows0.at[pl.ds(0, 125)], psum_h.at[pl.ds(ro, 125)])
        pltpu.sync_copy(cntsh.at[pl.ds(r0, 125)], ones.at[pl.ds(0, 125)])
        pltpu.sync_copy(ones.at[pl.ds(0, 125)], pcnt_h.at[pl.ds(ro, 125)])


def _agg_stage(x, src, dst):
    n_nodes = x.shape[0]
    k = pl.kernel(
        _agg_body,
        out_type=(
            jax.ShapeDtypeStruct((NC * n_nodes, 128), jnp.float32),
            jax.ShapeDtypeStruct((NC * n_nodes, 16), jnp.float32),
        ),
        mesh=plsc.VectorSubcoreMesh(core_axis_name="c", subcore_axis_name="s"),
        compiler_params=pltpu.CompilerParams(use_tc_tiling_on_sc=False),
        scratch_types=[
            pltpu.VMEM((CH,), jnp.int32),
            pltpu.VMEM((CH,), jnp.int32),
            pltpu.VMEM((CH,), jnp.int32),
            pltpu.VMEM((CH,), jnp.int32),
            pltpu.VMEM((CH, 128), jnp.float32),
            pltpu.VMEM((CH, 128), jnp.float32),
            pltpu.VMEM((CH, 16), jnp.float32),
            pltpu.VMEM_SHARED((n_nodes, 128), jnp.float32),
            pltpu.VMEM_SHARED((n_nodes, 16), jnp.float32),
            pltpu.SemaphoreType.DMA,
            pltpu.SemaphoreType.DMA,
        ],
    )
    return k(x, src, dst)


def _stage_b_body(psum_ref, pcnt_ref, gnnW_ref, gnnb_ref, wsrc_ref, wdst_ref,
                  zb_ref, psrc_ref, pdst_ref):
    n = psum_ref.shape[0] // 2
    summed = psum_ref[0:n] + psum_ref[n:2 * n]
    pcnt = pcnt_ref[0:n, 0:1] + pcnt_ref[n:2 * n, 0:1]
    cnt = jnp.maximum(pcnt, 1.0)
    agg = summed / cnt
    h = jnp.maximum(
        jnp.dot(agg, gnnW_ref[...].T, preferred_element_type=jnp.float32)
        + gnnb_ref[...], 0.0)
    psrc_ref[...] = (
        jnp.dot(h, wsrc_ref[...].T, preferred_element_type=jnp.float32)
        + zb_ref[...])
    pdst_ref[...] = jnp.dot(h, wdst_ref[...].T,
                            preferred_element_type=jnp.float32)


def _stage_b(psum, pcnt, gnn_W, gnn_b, wsrc, wdst, zb):
    n = psum.shape[0] // 2
    return pl.pallas_call(
        _stage_b_body,
        out_shape=(
            jax.ShapeDtypeStruct((n, 80), jnp.float32),
            jax.ShapeDtypeStruct((n, 80), jnp.float32),
        ),
    )(psum, pcnt, gnn_W, gnn_b, wsrc, wdst, zb)


def kernel(x_topology, x_temporal, edge_index, gnn_W, gnn_b, mean_W, mean_b,
           var_W, var_b, weight_W, weight_b):
    B, N, _ = x_topology.shape
    n_nodes = B * N
    x = jnp.concatenate([x_topology, x_temporal], axis=-1).reshape(
        n_nodes, -1).astype(jnp.float32)
    src = edge_index[0]
    dst = edge_index[1]

    psum, pcnt = _agg_stage(x, src, dst)

    wsrc = jnp.concatenate([mean_W[:, :128], var_W[:, :128],
                            jnp.tile(weight_W[:, :128], (16, 1))], axis=0)
    wdst = jnp.concatenate([mean_W[:, 128:], var_W[:, 128:],
                            jnp.tile(weight_W[:, 128:], (16, 1))], axis=0)
    zb = jnp.concatenate([mean_b, var_b, jnp.tile(weight_b, 16)])[None, :]

    psrc, pdst = _stage_b(psum, pcnt, gnn_W, gnn_b[None, :], wsrc, wdst, zb)
    mean, var, wt = _edge_stage(psrc, pdst, src, dst)
    return (mean, var, wt)
